# Initial kernel scaffold; baseline (speedup 1.0000x reference)
import functools

import jax
import jax.numpy as jnp
from jax import lax
from jax.experimental import pallas as pl
from jax.experimental.pallas import tpu as pltpu
from jax.experimental.pallas import tpu_sc as plsc

_N = 10000
_E = 320000
_D = 128
_H = 256
_C = 40
_QW = 48

_NC = 2
_NS = 16
_NW = _NC * _NS
_EPW = _E // _NW
_CHUNK = 125
_NCHUNK = _EPW // _CHUNK
_RPS = _N // _NS

_MESH = plsc.VectorSubcoreMesh(core_axis_name="c", subcore_axis_name="s")


def _sc_segsum_deg(x, src, dst, zeros_main, zeros_deg, ones):
    @functools.partial(
        pl.kernel,
        mesh=_MESH,
        out_type=[
            jax.ShapeDtypeStruct((_NC, _N, _D), jnp.float32),
            jax.ShapeDtypeStruct((_NC, _N, 16), jnp.float32),
        ],
        scratch_types=[
            pltpu.VMEM((_NCHUNK, _CHUNK), jnp.int32),
            pltpu.VMEM((_NCHUNK, _CHUNK), jnp.int32),
            pltpu.VMEM((_CHUNK, _D), jnp.float32),
            pltpu.VMEM((_CHUNK, 16), jnp.float32),
            pltpu.VMEM_SHARED((_N, _D), jnp.float32),
            pltpu.VMEM_SHARED((_N, 16), jnp.float32),
        ],
    )
    def k(x_hbm, src_hbm, dst_hbm, z_hbm, zd_hbm, ones_hbm, acc_out, deg_out,
          src_v, dst_v, rows_v, ones_v, acc_sh, deg_sh):
        c = lax.axis_index("c")
        s = lax.axis_index("s")
        wid = c * _NS + s
        rs = pl.ds(s * _RPS, _RPS)
        pltpu.sync_copy(z_hbm.at[rs], acc_sh.at[rs])
        pltpu.sync_copy(zd_hbm.at[rs], deg_sh.at[rs])
        pltpu.sync_copy(ones_hbm, ones_v)
        pltpu.sync_copy(src_hbm.at[wid], src_v)
        pltpu.sync_copy(dst_hbm.at[wid], dst_v)
        plsc.subcore_barrier()

        @pl.loop(0, _NCHUNK)
        def _(j):
            pltpu.sync_copy(x_hbm.at[src_v.at[j]], rows_v)
            pltpu.sync_copy(rows_v, acc_sh.at[dst_v.at[j]], add=True)
            pltpu.sync_copy(ones_v, deg_sh.at[dst_v.at[j]], add=True)

        plsc.subcore_barrier()
        pltpu.sync_copy(acc_sh.at[rs], acc_out.at[c].at[rs])
        pltpu.sync_copy(deg_sh.at[rs], deg_out.at[c].at[rs])

    return k(x, src, dst, zeros_main, zeros_deg, ones)


def _sc_segsum_q(q, src, dst, zeros_q):
    @functools.partial(
        pl.kernel,
        mesh=_MESH,
        out_type=jax.ShapeDtypeStruct((_NC, _N, _QW), jnp.float32),
        scratch_types=[
            pltpu.VMEM((_NCHUNK, _CHUNK), jnp.int32),
            pltpu.VMEM((_NCHUNK, _CHUNK), jnp.int32),
            pltpu.VMEM((_CHUNK, _QW), jnp.float32),
            pltpu.VMEM_SHARED((_N, _QW), jnp.float32),
        ],
    )
    def k(q_hbm, src_hbm, dst_hbm, z_hbm, acc_out, src_v, dst_v, rows_v, acc_sh):
        c = lax.axis_index("c")
        s = lax.axis_index("s")
        wid = c * _NS + s
        rs = pl.ds(s * _RPS, _RPS)
        pltpu.sync_copy(z_hbm.at[rs], acc_sh.at[rs])
        pltpu.sync_copy(src_hbm.at[wid], src_v)
        pltpu.sync_copy(dst_hbm.at[wid], dst_v)
        plsc.subcore_barrier()

        @pl.loop(0, _NCHUNK)
        def _(j):
            pltpu.sync_copy(q_hbm.at[src_v.at[j]], rows_v)
            pltpu.sync_copy(rows_v, acc_sh.at[dst_v.at[j]], add=True)

        plsc.subcore_barrier()
        pltpu.sync_copy(acc_sh.at[rs], acc_out.at[c].at[rs])

    return k(q, src, dst, zeros_q)


_BLK = 1000


def _tc_layer1(x, p, dg, w1r, w1n, b1, w2n_pad, w2r):
    def body(x_ref, p_ref, d_ref, w1r_ref, w1n_ref, b1_ref, w2n_ref, w2r_ref,
             q_ref, r_ref):
        deg = jnp.maximum(d_ref[0, :, 0:1] + d_ref[1, :, 0:1], 1.0)
        mean = (p_ref[0] + p_ref[1]) / deg
        h = jnp.maximum(
            jnp.dot(x_ref[...], w1r_ref[...], preferred_element_type=jnp.float32)
            + jnp.dot(mean, w1n_ref[...], preferred_element_type=jnp.float32)
            + b1_ref[...],
            0.0,
        )
        q_ref[...] = jnp.dot(h, w2n_ref[...], preferred_element_type=jnp.float32)
        r_ref[...] = jnp.dot(h, w2r_ref[...], preferred_element_type=jnp.float32)

    return pl.pallas_call(
        body,
        grid=(_N // _BLK,),
        in_specs=[
            pl.BlockSpec((_BLK, _D), lambda i: (i, 0)),
            pl.BlockSpec((_NC, _BLK, _D), lambda i: (0, i, 0)),
            pl.BlockSpec((_NC, _BLK, 16), lambda i: (0, i, 0)),
            pl.BlockSpec((_D, _H), lambda i: (0, 0)),
            pl.BlockSpec((_D, _H), lambda i: (0, 0)),
            pl.BlockSpec((1, _H), lambda i: (0, 0)),
            pl.BlockSpec((_H, _QW), lambda i: (0, 0)),
            pl.BlockSpec((_H, _C), lambda i: (0, 0)),
        ],
        out_specs=[
            pl.BlockSpec((_BLK, _QW), lambda i: (i, 0)),
            pl.BlockSpec((_BLK, _C), lambda i: (i, 0)),
        ],
        out_shape=[
            jax.ShapeDtypeStruct((_N, _QW), jnp.float32),
            jax.ShapeDtypeStruct((_N, _C), jnp.float32),
        ],
    )(x, p, dg, w1r, w1n, b1, w2n_pad, w2r)


def _tc_layer2(r, s2, dg, b2):
    def body(r_ref, s_ref, d_ref, b2_ref, o_ref):
        deg = jnp.maximum(d_ref[0, :, 0:1] + d_ref[1, :, 0:1], 1.0)
        mean = (s_ref[0, :, :_C] + s_ref[1, :, :_C]) / deg
        o_ref[...] = r_ref[...] + mean + b2_ref[...]

    return pl.pallas_call(
        body,
        grid=(_N // _BLK,),
        in_specs=[
            pl.BlockSpec((_BLK, _C), lambda i: (i, 0)),
            pl.BlockSpec((_NC, _BLK, _QW), lambda i: (0, i, 0)),
            pl.BlockSpec((_NC, _BLK, 16), lambda i: (0, i, 0)),
            pl.BlockSpec((1, _C), lambda i: (0, 0)),
        ],
        out_specs=pl.BlockSpec((_BLK, _C), lambda i: (i, 0)),
        out_shape=jax.ShapeDtypeStruct((_N, _C), jnp.float32),
    )(r, s2, dg, b2)


def kernel(x, edge_index, W1_root, W1_neigh, b1, W2_root, W2_neigh, b2):
    src = edge_index[0].reshape(_NW, _NCHUNK, _CHUNK)
    dst = edge_index[1].reshape(_NW, _NCHUNK, _CHUNK)
    zeros_main = jnp.zeros((_N, _D), jnp.float32)
    zeros_deg = jnp.zeros((_N, 16), jnp.float32)
    zeros_q = jnp.zeros((_N, _QW), jnp.float32)
    ones = jnp.ones((_CHUNK, 16), jnp.float32)

    p1, dg = _sc_segsum_deg(x, src, dst, zeros_main, zeros_deg, ones)
    q, r = _tc_layer1(
        x, p1, dg, W1_root, W1_neigh, b1.reshape(1, _H),
        jnp.pad(W2_neigh, ((0, 0), (0, _QW - _C))), W2_root,
    )
    p2 = _sc_segsum_q(q, src, dst, zeros_q)
    out = _tc_layer2(r, p2, dg, b2.reshape(1, _C))
    return out


# speedup vs baseline: 11.5627x; 11.5627x over previous
import functools

import jax
import jax.numpy as jnp
from jax import lax
from jax.experimental import pallas as pl
from jax.experimental.pallas import tpu as pltpu
from jax.experimental.pallas import tpu_sc as plsc

_N = 10000
_E = 320000
_D = 128
_H = 256
_C = 40
_QW = 48

_NC = 2
_NS = 16
_NW = _NC * _NS
_EPW = _E // _NW
_CHUNK = 125
_NCHUNK = _EPW // _CHUNK
_NP = 10240
_RPS = _NP // _NS

_MESH = plsc.VectorSubcoreMesh(core_axis_name="c", subcore_axis_name="s")


def _sc_segsum(x, src, dst, zeros, width):
    @functools.partial(
        pl.kernel,
        mesh=_MESH,
        out_type=jax.ShapeDtypeStruct((_NC, _NP, width), jnp.float32),
        compiler_params=pltpu.CompilerParams(use_tc_tiling_on_sc=False),
        scratch_types=[
            pltpu.VMEM((_NCHUNK, _CHUNK), jnp.int32),
            pltpu.VMEM((_NCHUNK, _CHUNK), jnp.int32),
            pltpu.VMEM((_CHUNK, width), jnp.float32),
            pltpu.VMEM_SHARED((_NP, width), jnp.float32),
        ],
    )
    def k(x_hbm, src_hbm, dst_hbm, z_hbm, acc_out, src_v, dst_v, rows_v, acc_sh):
        c = lax.axis_index("c")
        s = lax.axis_index("s")
        wid = c * _NS + s
        rs = pl.ds(s * _RPS, _RPS)
        pltpu.sync_copy(z_hbm.at[rs], acc_sh.at[rs])
        pltpu.sync_copy(src_hbm.at[wid], src_v)
        pltpu.sync_copy(dst_hbm.at[wid], dst_v)
        plsc.subcore_barrier()

        @pl.loop(0, _NCHUNK)
        def _(j):
            pltpu.sync_copy(x_hbm.at[src_v.at[j]], rows_v)
            pltpu.sync_copy(rows_v, acc_sh.at[dst_v.at[j]], add=True)

        plsc.subcore_barrier()
        pltpu.sync_copy(acc_sh.at[rs], acc_out.at[c].at[rs])

    return k(x, src, dst, zeros)


def _sc_deg(dst, zeros_deg, ones):
    @functools.partial(
        pl.kernel,
        mesh=_MESH,
        out_type=jax.ShapeDtypeStruct((_NC, _NP, 16), jnp.float32),
        compiler_params=pltpu.CompilerParams(use_tc_tiling_on_sc=False),
        scratch_types=[
            pltpu.VMEM((_NCHUNK, _CHUNK), jnp.int32),
            pltpu.VMEM((_CHUNK, 16), jnp.float32),
            pltpu.VMEM_SHARED((_NP, 16), jnp.float32),
        ],
    )
    def k(dst_hbm, z_hbm, ones_hbm, deg_out, dst_v, ones_v, deg_sh):
        c = lax.axis_index("c")
        s = lax.axis_index("s")
        wid = c * _NS + s
        rs = pl.ds(s * _RPS, _RPS)
        pltpu.sync_copy(z_hbm.at[rs], deg_sh.at[rs])
        pltpu.sync_copy(ones_hbm, ones_v)
        pltpu.sync_copy(dst_hbm.at[wid], dst_v)
        plsc.subcore_barrier()

        @pl.loop(0, _NCHUNK)
        def _(j):
            pltpu.sync_copy(ones_v, deg_sh.at[dst_v.at[j]], add=True)

        plsc.subcore_barrier()
        pltpu.sync_copy(deg_sh.at[rs], deg_out.at[c].at[rs])

    return k(dst, zeros_deg, ones)


_BLK = 1000


def _tc_layer1(x, p, dg, w1r, w1n, b1, w2n_pad, w2r):
    def body(x_ref, p_ref, d_ref, w1r_ref, w1n_ref, b1_ref, w2n_ref, w2r_ref,
             q_ref, r_ref):
        deg = jnp.maximum(d_ref[0, :, 0:1] + d_ref[1, :, 0:1], 1.0)
        mean = (p_ref[0] + p_ref[1]) / deg
        h = jnp.maximum(
            jnp.dot(x_ref[...], w1r_ref[...], preferred_element_type=jnp.float32)
            + jnp.dot(mean, w1n_ref[...], preferred_element_type=jnp.float32)
            + b1_ref[...],
            0.0,
        )
        q_ref[...] = jnp.dot(h, w2n_ref[...], preferred_element_type=jnp.float32)
        r_ref[...] = jnp.dot(h, w2r_ref[...], preferred_element_type=jnp.float32)

    return pl.pallas_call(
        body,
        grid=(_N // _BLK,),
        in_specs=[
            pl.BlockSpec((_BLK, _D), lambda i: (i, 0)),
            pl.BlockSpec((_NC, _BLK, _D), lambda i: (0, i, 0)),
            pl.BlockSpec((_NC, _BLK, 16), lambda i: (0, i, 0)),
            pl.BlockSpec((_D, _H), lambda i: (0, 0)),
            pl.BlockSpec((_D, _H), lambda i: (0, 0)),
            pl.BlockSpec((1, _H), lambda i: (0, 0)),
            pl.BlockSpec((_H, _QW), lambda i: (0, 0)),
            pl.BlockSpec((_H, _C), lambda i: (0, 0)),
        ],
        out_specs=[
            pl.BlockSpec((_BLK, _QW), lambda i: (i, 0)),
            pl.BlockSpec((_BLK, _C), lambda i: (i, 0)),
        ],
        out_shape=[
            jax.ShapeDtypeStruct((_N, _QW), jnp.float32),
            jax.ShapeDtypeStruct((_N, _C), jnp.float32),
        ],
    )(x, p, dg, w1r, w1n, b1, w2n_pad, w2r)


def _tc_layer2(r, s2, dg, b2):
    def body(r_ref, s_ref, d_ref, b2_ref, o_ref):
        deg = jnp.maximum(d_ref[0, :, 0:1] + d_ref[1, :, 0:1], 1.0)
        mean = (s_ref[0, :, :_C] + s_ref[1, :, :_C]) / deg
        o_ref[...] = r_ref[...] + mean + b2_ref[...]

    return pl.pallas_call(
        body,
        grid=(_N // _BLK,),
        in_specs=[
            pl.BlockSpec((_BLK, _C), lambda i: (i, 0)),
            pl.BlockSpec((_NC, _BLK, _QW), lambda i: (0, i, 0)),
            pl.BlockSpec((_NC, _BLK, 16), lambda i: (0, i, 0)),
            pl.BlockSpec((1, _C), lambda i: (0, 0)),
        ],
        out_specs=pl.BlockSpec((_BLK, _C), lambda i: (i, 0)),
        out_shape=jax.ShapeDtypeStruct((_N, _C), jnp.float32),
    )(r, s2, dg, b2)


def kernel(x, edge_index, W1_root, W1_neigh, b1, W2_root, W2_neigh, b2):
    src = edge_index[0].reshape(_NW, _NCHUNK, _CHUNK)
    dst = edge_index[1].reshape(_NW, _NCHUNK, _CHUNK)
    zeros_main = jnp.zeros((_NP, _D), jnp.float32)
    zeros_deg = jnp.zeros((_NP, 16), jnp.float32)
    zeros_q = jnp.zeros((_NP, _QW), jnp.float32)
    ones = jnp.ones((_CHUNK, 16), jnp.float32)

    dg = _sc_deg(dst, zeros_deg, ones)
    p1 = _sc_segsum(x, src, dst, zeros_main, _D)
    q, r = _tc_layer1(
        x, p1, dg, W1_root, W1_neigh, b1.reshape(1, _H),
        jnp.pad(W2_neigh, ((0, 0), (0, _QW - _C))), W2_root,
    )
    p2 = _sc_segsum(q, src, dst, zeros_q, _QW)
    out = _tc_layer2(r, p2, dg, b2.reshape(1, _C))
    return out
